# baseline (speedup 1.0000x reference)
import functools

import jax
import jax.numpy as jnp
from jax import lax
from jax.experimental import pallas as pl
from jax.experimental.pallas import tpu as pltpu
from jax.experimental.pallas import tpu_sc as plsc

NC = 2
NS = 16
CHUNK = 128


def _sc_aggregate(n_rows, nch, dh):
    rpt = n_rows // NS
    nsup = nch // 2

    mesh = plsc.VectorSubcoreMesh(core_axis_name="c", subcore_axis_name="s")

    @functools.partial(
        pl.kernel,
        out_type=(
            jax.ShapeDtypeStruct((NC, n_rows, dh), jnp.float32),
            jax.ShapeDtypeStruct((NC, n_rows, 16), jnp.float32),
        ),
        mesh=mesh,
        scratch_types=[
            pltpu.VMEM((2, 2, CHUNK), jnp.int32),
            pltpu.VMEM((2, 2, CHUNK), jnp.int32),
            pltpu.VMEM((CHUNK, dh), jnp.float32),
            pltpu.VMEM((CHUNK, dh), jnp.float32),
            pltpu.VMEM((CHUNK, 16), jnp.float32),
            pltpu.VMEM_SHARED((n_rows, dh), jnp.float32),
            pltpu.VMEM_SHARED((n_rows, dh), jnp.float32),
            pltpu.VMEM_SHARED((n_rows, 16), jnp.float32),
            pltpu.SemaphoreType.DMA,
            pltpu.SemaphoreType.DMA,
            pltpu.SemaphoreType.DMA,
        ],
        compiler_params=pltpu.CompilerParams(use_tc_tiling_on_sc=False),
    )
    def sc_kernel(xp_hbm, src_hbm, dst_hbm, zf_hbm, zd_hbm, ones_hbm,
                  agg_out, deg_out,
                  src_v, dst_v, buf0, buf1, ones_v, x_sh, agg_sh, deg_sh,
                  gsem0, gsem1, isem):
        cid = lax.axis_index("c")
        sid = lax.axis_index("s")
        base = sid * rpt

        pltpu.sync_copy(xp_hbm.at[cid, pl.ds(base, rpt)],
                        x_sh.at[pl.ds(base, rpt)])
        pltpu.sync_copy(zf_hbm.at[pl.ds(base, rpt)], agg_sh.at[pl.ds(base, rpt)])
        pltpu.sync_copy(zd_hbm.at[pl.ds(base, rpt)], deg_sh.at[pl.ds(base, rpt)])
        pltpu.sync_copy(ones_hbm, ones_v)
        pltpu.sync_copy(src_hbm.at[sid, pl.ds(0, 2)], src_v.at[0])
        pltpu.sync_copy(dst_hbm.at[sid, pl.ds(0, 2)], dst_v.at[0])
        plsc.subcore_barrier()

        def body(i, carry):
            slot = lax.rem(i, 2)
            nxt = lax.rem(i + 1, 2)

            @pl.when(i + 1 < nsup)
            def _():
                pltpu.async_copy(src_hbm.at[sid, pl.ds(2 * i + 2, 2)],
                                 src_v.at[nxt], isem)
                pltpu.async_copy(dst_hbm.at[sid, pl.ds(2 * i + 2, 2)],
                                 dst_v.at[nxt], isem)

            pltpu.async_copy(x_sh.at[src_v.at[slot, 0]], buf0, gsem0)
            pltpu.async_copy(x_sh.at[src_v.at[slot, 1]], buf1, gsem1)
            pltpu.make_async_copy(x_sh.at[src_v.at[slot, 0]], buf0, gsem0).wait()
            pltpu.sync_copy(buf0, agg_sh.at[dst_v.at[slot, 0]], add=True)

            @pl.when(cid == 0)
            def _():
                pltpu.sync_copy(ones_v, deg_sh.at[dst_v.at[slot, 0]], add=True)

            pltpu.make_async_copy(x_sh.at[src_v.at[slot, 1]], buf1, gsem1).wait()
            pltpu.sync_copy(buf1, agg_sh.at[dst_v.at[slot, 1]], add=True)

            @pl.when(cid == 1)
            def _():
                pltpu.sync_copy(ones_v, deg_sh.at[dst_v.at[slot, 1]], add=True)

            @pl.when(i + 1 < nsup)
            def _():
                pltpu.make_async_copy(src_hbm.at[sid, pl.ds(2 * i + 2, 2)],
                                      src_v.at[nxt], isem).wait()
                pltpu.make_async_copy(dst_hbm.at[sid, pl.ds(2 * i + 2, 2)],
                                      dst_v.at[nxt], isem).wait()

            return carry

        lax.fori_loop(0, nsup, body, 0, unroll=False)

        plsc.subcore_barrier()
        pltpu.sync_copy(agg_sh.at[pl.ds(base, rpt)],
                        agg_out.at[cid, pl.ds(base, rpt)])
        pltpu.sync_copy(deg_sh.at[pl.ds(base, rpt)],
                        deg_out.at[cid, pl.ds(base, rpt)])

    return sc_kernel


def _tc_transform(n_nodes, d_feat, block_rows):
    def body(x_ref, p_ref, dp_ref, wn_ref, ws_ref, b_ref, out_ref):
        agg = jnp.concatenate([p_ref[0], p_ref[1]], axis=1)
        deg = dp_ref[0, :, 0:1] + dp_ref[1, :, 0:1]
        agg = agg / jnp.maximum(deg, 1.0)
        acc = jnp.dot(agg, wn_ref[...], preferred_element_type=jnp.float32)
        acc += jnp.dot(x_ref[...], ws_ref[...], preferred_element_type=jnp.float32)
        out_ref[...] = jnp.maximum(acc + b_ref[...], 0.0)

    grid = n_nodes // block_rows
    dh = d_feat // 2
    return pl.pallas_call(
        body,
        grid=(grid,),
        in_specs=[
            pl.BlockSpec((block_rows, d_feat), lambda i: (i, 0)),
            pl.BlockSpec((NC, block_rows, dh), lambda i: (0, i, 0)),
            pl.BlockSpec((NC, block_rows, 16), lambda i: (0, i, 0)),
            pl.BlockSpec((d_feat, d_feat), lambda i: (0, 0)),
            pl.BlockSpec((d_feat, d_feat), lambda i: (0, 0)),
            pl.BlockSpec((1, d_feat), lambda i: (0, 0)),
        ],
        out_specs=pl.BlockSpec((block_rows, d_feat), lambda i: (i, 0)),
        out_shape=jax.ShapeDtypeStruct((n_nodes, d_feat), jnp.float32),
        compiler_params=pltpu.CompilerParams(
            dimension_semantics=("arbitrary",),
        ),
    )


def kernel(x, edge_index, W_self, W_neigh, b):
    n, d = x.shape
    dh = d // 2
    e = edge_index.shape[1]

    nch = -(-e // (NS * CHUNK))
    nch += nch % 2
    e_pad = NS * CHUNK * nch
    src = jnp.concatenate(
        [edge_index[0], jnp.zeros((e_pad - e,), jnp.int32)]).reshape(NS, nch, CHUNK)
    dst = jnp.concatenate(
        [edge_index[1], jnp.full((e_pad - e,), n, jnp.int32)]).reshape(NS, nch, CHUNK)

    rpt = 8 * (-(-(n + 1) // (NS * 8)))
    n_rows = NS * rpt

    xpad = jnp.pad(x, ((0, n_rows - n), (0, 0)))
    xp = jnp.stack([xpad[:, :dh], xpad[:, dh:]])

    zeros_feat = jnp.zeros((n_rows, dh), jnp.float32)
    zeros_deg = jnp.zeros((n_rows, 16), jnp.float32)
    ones = jnp.ones((CHUNK, 16), jnp.float32)

    agg_p, deg_p = _sc_aggregate(n_rows, nch, dh)(
        xp, src, dst, zeros_feat, zeros_deg, ones)

    block_rows = 1000 if n % 1000 == 0 else 8
    out = _tc_transform(n, d, block_rows)(
        x, agg_p, deg_p, W_neigh, W_self, b.reshape(1, d))
    return out
